# baseline (speedup 1.0000x reference)
import functools

import jax
import jax.numpy as jnp
from jax import lax
from jax.experimental import pallas as pl
from jax.experimental.pallas import tpu as pltpu
from jax.experimental.pallas import tpu_sc as plsc

N_ = 10000
E_ = 320000
F_ = 128
ED_ = 16
H_ = 64
D_ = 128
PAD_ = D_ - H_ - 3

NC_ = 2
NS_ = 16
NW_ = NC_ * NS_

EC_ = (166400, 153600)
GK_ = 200
SK_ = 80

NP_ = 10240
RPT_ = NP_ // NS_
ZR_ = 64

NBLK_ = 1000
EBLK_ = 3200

assert sum(EC_) == E_
for _e in EC_:
    assert _e % (NW_ * GK_) == 0 or (_e // NW_) % GK_ == 0
    assert (_e // NW_) % SK_ == 0 and _e % EBLK_ == 0
assert GK_ % 8 == 0 and SK_ % 8 == 0
assert RPT_ % ZR_ == 0 and ZR_ % 8 == 0 and NP_ % NS_ == 0
assert N_ % NBLK_ == 0 and D_ % 128 == 0


def _tables_body(h_ref, x_ref, whr_ref, whc_ref, be1_ref, ta_ref, tb_ref):
    h = h_ref[...]
    x3 = x_ref[...]
    z = jnp.zeros((h.shape[0], PAD_), jnp.float32)
    a = jnp.dot(h, whr_ref[...], preferred_element_type=jnp.float32) + be1_ref[...]
    b = jnp.dot(h, whc_ref[...], preferred_element_type=jnp.float32)
    ta_ref[...] = jnp.concatenate([a, x3, z], axis=1).astype(jnp.bfloat16)
    tb_ref[...] = jnp.concatenate([b, -x3, z], axis=1).astype(jnp.bfloat16)


def _tables(h, x, whr, whc, be1r):
    g = N_ // NBLK_
    return pl.pallas_call(
        _tables_body,
        grid=(g,),
        in_specs=[
            pl.BlockSpec((NBLK_, F_), lambda i: (i, 0)),
            pl.BlockSpec((NBLK_, 3), lambda i: (i, 0)),
            pl.BlockSpec((F_, H_), lambda i: (0, 0)),
            pl.BlockSpec((F_, H_), lambda i: (0, 0)),
            pl.BlockSpec((1, H_), lambda i: (0, 0)),
        ],
        out_specs=[
            pl.BlockSpec((NBLK_, D_), lambda i: (i, 0)),
            pl.BlockSpec((NBLK_, D_), lambda i: (i, 0)),
        ],
        out_shape=[
            jax.ShapeDtypeStruct((N_, D_), jnp.bfloat16),
            jax.ShapeDtypeStruct((N_, D_), jnp.bfloat16),
        ],
    )(h, x, whr, whc, be1r)


def _ring(nch, chunk_step):
    def group(g, carry):
        chunk_step(g * 2, 0)
        chunk_step(g * 2 + 1, 1)
        return carry

    lax.fori_loop(0, nch // 2, group, 0)
    if nch % 2:
        chunk_step(nch - 1, 0)


@functools.lru_cache(maxsize=None)
def _sc_kernels(e_chunk):
    epw = e_chunk // NW_
    gnc = epw // GK_
    snc = epw // SK_
    mesh = plsc.VectorSubcoreMesh(core_axis_name="c", subcore_axis_name="s",
                                  num_cores=NC_, num_subcores=NS_)

    @functools.partial(
        pl.kernel,
        out_type=jax.ShapeDtypeStruct((e_chunk, D_), jnp.bfloat16),
        mesh=mesh,
        compiler_params=pltpu.CompilerParams(use_tc_tiling_on_sc=False),
        scratch_types=[
            pltpu.VMEM((epw,), jnp.int32),
            pltpu.VMEM((epw,), jnp.int32),
            pltpu.VMEM((GK_, D_), jnp.bfloat16),
            pltpu.VMEM((GK_, D_), jnp.bfloat16),
            pltpu.VMEM((GK_, D_), jnp.bfloat16),
            pltpu.VMEM((GK_, D_), jnp.bfloat16),
            [pltpu.SemaphoreType.DMA] * 6,
        ],
    )
    def _sc_gather(ta, tb, row, col, gsum, idx1, idx2,
                   bufa0, bufa1, bufb0, bufb1, sems):
        wid = lax.axis_index("s") * NC_ + lax.axis_index("c")
        base = wid * epw

        pltpu.sync_copy(row.at[pl.ds(base, epw)], idx1)
        pltpu.sync_copy(col.at[pl.ds(base, epw)], idx2)

        bufsa = (bufa0, bufa1)
        bufsb = (bufb0, bufb1)

        def g_copies(c, b):
            ia = idx1.at[pl.ds(c * GK_, GK_)]
            ib = idx2.at[pl.ds(c * GK_, GK_)]
            return (pltpu.make_async_copy(ta.at[ia], bufsa[b], sems[b]),
                    pltpu.make_async_copy(tb.at[ib], bufsb[b], sems[2 + b]))

        def w_copy(c, b):
            dst = pl.ds(base + c * GK_, GK_)
            return pltpu.make_async_copy(bufsa[b], gsum.at[dst], sems[4 + b])

        def start(copies):
            for cp in copies:
                cp.start()

        def wait(copies):
            for cp in copies:
                cp.wait()

        start(g_copies(0, 0))

        nlane = D_ // 32

        def chunk_step(c, b):
            nb = 1 - b

            @pl.when(c >= 1)
            def _():
                w_copy(c - 1, nb).wait()

            @pl.when(c + 1 < gnc)
            def _():
                start(g_copies(c + 1, nb))

            wait(g_copies(c, b))

            ba = bufsa[b]
            bb = bufsb[b]

            def add_row(r, carry):
                for l in range(nlane):
                    sl = pl.ds(l * 32, 32)
                    ba[r, sl] = ba[r, sl] + bb[r, sl]
                return carry

            lax.fori_loop(0, GK_, add_row, 0)
            w_copy(c, b).start()

        _ring(gnc, chunk_step)
        w_copy(gnc - 1, (gnc - 1) % 2).wait()

    @functools.partial(
        pl.kernel,
        out_type=jax.ShapeDtypeStruct((NC_, NP_, D_), jnp.float32),
        mesh=mesh,
        scratch_types=[
            pltpu.VMEM((snc, SK_), jnp.int32),
            pltpu.VMEM((SK_, D_), jnp.float32),
            pltpu.VMEM((SK_, D_), jnp.float32),
            pltpu.VMEM((ZR_, D_), jnp.float32),
            pltpu.VMEM_SHARED((NP_, D_), jnp.float32),
            [pltpu.SemaphoreType.DMA] * 4,
        ],
    )
    def _sc_scatter(vals, row2d, out, idx2d, buf0, buf1, zbuf, acc, sems):
        cid = lax.axis_index("c")
        sid = lax.axis_index("s")
        wid = sid * NC_ + cid

        pltpu.sync_copy(row2d.at[wid], idx2d)

        zv = jnp.zeros((16,), jnp.float32)
        nlane = D_ // 16

        def zb(i, carry):
            r = i // nlane
            c = i % nlane
            zbuf[r, pl.ds(c * 16, 16)] = zv
            return carry

        lax.fori_loop(0, ZR_ * nlane, zb, 0)

        def zcopy(j, carry):
            pltpu.sync_copy(zbuf, acc.at[pl.ds(sid * RPT_ + j * ZR_, ZR_)])
            return carry

        lax.fori_loop(0, RPT_ // ZR_, zcopy, 0)
        plsc.subcore_barrier()

        base = wid * epw
        bufs = (buf0, buf1)

        def r_copy(c, b):
            src = vals.at[pl.ds(base + c * SK_, SK_)]
            return pltpu.make_async_copy(src, bufs[b], sems[b])

        def a_copy(c, b):
            return pltpu.make_async_copy(bufs[b], acc.at[idx2d.at[c]],
                                         sems[2 + b])

        r_copy(0, 0).start()

        def chunk_step(c, b):
            nb = 1 - b

            @pl.when(c >= 1)
            def _():
                a_copy(c - 1, nb).wait()

            @pl.when(c + 1 < snc)
            def _():
                r_copy(c + 1, nb).start()

            r_copy(c, b).wait()
            a_copy(c, b).start(add=True)

        _ring(snc, chunk_step)
        a_copy(snc - 1, (snc - 1) % 2).wait()
        plsc.subcore_barrier()

        def drain(j, carry):
            r0 = sid * RPT_ + j * ZR_
            pltpu.sync_copy(acc.at[pl.ds(r0, ZR_)], zbuf)
            pltpu.sync_copy(zbuf, out.at[cid, pl.ds(r0, ZR_)])
            return carry

        lax.fori_loop(0, RPT_ // ZR_, drain, 0)

    return _sc_gather, _sc_scatter


def _edge_body(g_ref, ea_ref, wea_ref, wd_ref, we2_ref, be2_ref,
               wc_ref, bc_ref, out_ref):
    g1 = g_ref[...].astype(jnp.float32)
    gm = g1[:, :H_]
    rel = g1[:, H_:H_ + 3]
    d2 = jnp.sum(rel * rel, axis=1, keepdims=True)
    pre = gm + jnp.dot(ea_ref[...], wea_ref[...],
                       preferred_element_type=jnp.float32) + d2 * wd_ref[...]
    m1 = pre * jax.nn.sigmoid(pre)
    m2v = jnp.dot(m1, we2_ref[...], preferred_element_type=jnp.float32) + be2_ref[...]
    m2 = m2v * jax.nn.sigmoid(m2v)
    coef = jnp.tanh(jnp.dot(m2, wc_ref[...],
                            preferred_element_type=jnp.float32) + bc_ref[...])
    z = jnp.zeros((g1.shape[0], PAD_), jnp.float32)
    out_ref[...] = jnp.concatenate([m2, rel * coef, z], axis=1)


def _edges(g, ea, wea, wdr, we2, be2r, wc, bcr):
    e = g.shape[0]
    return pl.pallas_call(
        _edge_body,
        grid=(e // EBLK_,),
        in_specs=[
            pl.BlockSpec((EBLK_, D_), lambda i: (i, 0)),
            pl.BlockSpec((EBLK_, ED_), lambda i: (i, 0)),
            pl.BlockSpec((ED_, H_), lambda i: (0, 0)),
            pl.BlockSpec((1, H_), lambda i: (0, 0)),
            pl.BlockSpec((H_, H_), lambda i: (0, 0)),
            pl.BlockSpec((1, H_), lambda i: (0, 0)),
            pl.BlockSpec((H_, 1), lambda i: (0, 0)),
            pl.BlockSpec((1, 1), lambda i: (0, 0)),
        ],
        out_specs=pl.BlockSpec((EBLK_, D_), lambda i: (i, 0)),
        out_shape=jax.ShapeDtypeStruct((e, D_), jnp.float32),
    )(g, ea, wea, wdr, we2, be2r, wc, bcr)


def _node_body(h_ref, x_ref, p00_ref, p01_ref, p10_ref, p11_ref,
               wn1h_ref, wn1m_ref, bn1_ref, wn2_ref, bn2_ref,
               hn_ref, xn_ref):
    h = h_ref[...]
    p = (p00_ref[...] + p01_ref[...]) + (p10_ref[...] + p11_ref[...])
    magg = p[:, :H_]
    dx = p[:, H_:H_ + 3]
    t = jnp.dot(h, wn1h_ref[...], preferred_element_type=jnp.float32)
    t = t + jnp.dot(magg, wn1m_ref[...], preferred_element_type=jnp.float32)
    t = t + bn1_ref[...]
    t = t * jax.nn.sigmoid(t)
    hn_ref[...] = h + jnp.dot(t, wn2_ref[...],
                              preferred_element_type=jnp.float32) + bn2_ref[...]
    xn_ref[...] = x_ref[...] + dx


def _nodes(h, x, parts, wn1h, wn1m, bn1r, wn2, bn2r):
    g = N_ // NBLK_
    pspec = pl.BlockSpec((NBLK_, D_), lambda i: (i, 0))
    return pl.pallas_call(
        _node_body,
        grid=(g,),
        in_specs=[
            pl.BlockSpec((NBLK_, F_), lambda i: (i, 0)),
            pl.BlockSpec((NBLK_, 3), lambda i: (i, 0)),
            pspec, pspec, pspec, pspec,
            pl.BlockSpec((F_, H_), lambda i: (0, 0)),
            pl.BlockSpec((H_, H_), lambda i: (0, 0)),
            pl.BlockSpec((1, H_), lambda i: (0, 0)),
            pl.BlockSpec((H_, F_), lambda i: (0, 0)),
            pl.BlockSpec((1, F_), lambda i: (0, 0)),
        ],
        out_specs=[
            pl.BlockSpec((NBLK_, F_), lambda i: (i, 0)),
            pl.BlockSpec((NBLK_, 3), lambda i: (i, 0)),
        ],
        out_shape=[
            jax.ShapeDtypeStruct((N_, F_), jnp.float32),
            jax.ShapeDtypeStruct((N_, 3), jnp.float32),
        ],
    )(h, x, *parts, wn1h, wn1m, bn1r, wn2, bn2r)


def kernel(h, x, edge_index, edge_attr, We1, be1, We2, be2, Wc, bc,
           Wn1, bn1, Wn2, bn2):
    row = edge_index[0]
    col = edge_index[1]
    whr = We1[:F_]
    whc = We1[F_:2 * F_]
    wdr = We1[2 * F_:2 * F_ + 1]
    wea = We1[2 * F_ + 1:]
    be2r = be2.reshape(1, H_)
    bcr = bc.reshape(1, 1)

    ta, tb = _tables(h, x, whr, whc, be1.reshape(1, H_))

    parts = []
    off = 0
    for e_chunk in EC_:
        sc_gather, sc_scatter = _sc_kernels(e_chunk)
        rowc = lax.slice(row, (off,), (off + e_chunk,))
        colc = lax.slice(col, (off,), (off + e_chunk,))
        eac = lax.slice(edge_attr, (off, 0), (off + e_chunk, ED_))
        gsum = sc_gather(ta, tb, rowc, colc)
        vals = _edges(gsum, eac, wea, wdr, We2, be2r, Wc, bcr)
        snc = e_chunk // NW_ // SK_
        pc = sc_scatter(vals, rowc.reshape(NW_, snc, SK_))
        parts.extend([pc[0, :N_], pc[1, :N_]])
        off += e_chunk

    return _nodes(h, x, parts, Wn1[:F_], Wn1[F_:],
                  bn1.reshape(1, H_), Wn2, bn2.reshape(1, F_))


# speedup vs baseline: 1.4424x; 1.4424x over previous
import functools

import jax
import jax.numpy as jnp
from jax import lax
from jax.experimental import pallas as pl
from jax.experimental.pallas import tpu as pltpu
from jax.experimental.pallas import tpu_sc as plsc

N_ = 10000
E_ = 320000
F_ = 128
ED_ = 16
H_ = 64
D_ = 128
PAD_ = D_ - H_ - 3

NC_ = 2
NS_ = 16
NW_ = NC_ * NS_

EC_ = (166400, 153600)
GK_ = 200
SK_ = 80

NP_ = 10240
RPT_ = NP_ // NS_
ZR_ = 64

NBLK_ = 1000
EBLK_ = 3200

assert sum(EC_) == E_
for _e in EC_:
    assert _e % (NW_ * GK_) == 0 or (_e // NW_) % GK_ == 0
    assert (_e // NW_) % SK_ == 0 and _e % EBLK_ == 0
assert GK_ % 8 == 0 and SK_ % 8 == 0
assert RPT_ % ZR_ == 0 and ZR_ % 8 == 0 and NP_ % NS_ == 0
assert N_ % NBLK_ == 0 and D_ % 128 == 0


def _tables_body(h_ref, x_ref, whr_ref, whc_ref, be1_ref, ta_ref, tb_ref):
    h = h_ref[...]
    x3 = x_ref[...]
    z = jnp.zeros((h.shape[0], PAD_), jnp.float32)
    a = jnp.dot(h, whr_ref[...], preferred_element_type=jnp.float32) + be1_ref[...]
    b = jnp.dot(h, whc_ref[...], preferred_element_type=jnp.float32)
    ta_ref[...] = jnp.concatenate([a, x3, z], axis=1)
    tb_ref[...] = jnp.concatenate([b, -x3, z], axis=1)


def _tables(h, x, whr, whc, be1r):
    g = N_ // NBLK_
    return pl.pallas_call(
        _tables_body,
        grid=(g,),
        in_specs=[
            pl.BlockSpec((NBLK_, F_), lambda i: (i, 0)),
            pl.BlockSpec((NBLK_, 3), lambda i: (i, 0)),
            pl.BlockSpec((F_, H_), lambda i: (0, 0)),
            pl.BlockSpec((F_, H_), lambda i: (0, 0)),
            pl.BlockSpec((1, H_), lambda i: (0, 0)),
        ],
        out_specs=[
            pl.BlockSpec((NBLK_, D_), lambda i: (i, 0)),
            pl.BlockSpec((NBLK_, D_), lambda i: (i, 0)),
        ],
        out_shape=[
            jax.ShapeDtypeStruct((N_, D_), jnp.float32),
            jax.ShapeDtypeStruct((N_, D_), jnp.float32),
        ],
    )(h, x, whr, whc, be1r)


def _ring(nch, chunk_step):
    def group(g, carry):
        chunk_step(g * 2, 0)
        chunk_step(g * 2 + 1, 1)
        return carry

    lax.fori_loop(0, nch // 2, group, 0)
    if nch % 2:
        chunk_step(nch - 1, 0)


@functools.lru_cache(maxsize=None)
def _sc_kernels(e_chunk):
    epw = e_chunk // NW_
    gnc = epw // GK_
    snc = epw // SK_
    mesh = plsc.VectorSubcoreMesh(core_axis_name="c", subcore_axis_name="s",
                                  num_cores=NC_, num_subcores=NS_)

    @functools.partial(
        pl.kernel,
        out_type=jax.ShapeDtypeStruct((e_chunk, D_), jnp.float32),
        mesh=mesh,
        scratch_types=[
            pltpu.VMEM((epw,), jnp.int32),
            pltpu.VMEM((epw,), jnp.int32),
            pltpu.VMEM((GK_, D_), jnp.float32),
            pltpu.VMEM((GK_, D_), jnp.float32),
            pltpu.VMEM((GK_, D_), jnp.float32),
            pltpu.VMEM((GK_, D_), jnp.float32),
            [pltpu.SemaphoreType.DMA] * 6,
        ],
    )
    def _sc_gather(ta, tb, row, col, gsum, idx1, idx2,
                   bufa0, bufa1, bufb0, bufb1, sems):
        wid = lax.axis_index("s") * NC_ + lax.axis_index("c")
        base = wid * epw

        pltpu.sync_copy(row.at[pl.ds(base, epw)], idx1)
        pltpu.sync_copy(col.at[pl.ds(base, epw)], idx2)

        bufsa = (bufa0, bufa1)
        bufsb = (bufb0, bufb1)

        def g_copies(c, b):
            ia = idx1.at[pl.ds(c * GK_, GK_)]
            ib = idx2.at[pl.ds(c * GK_, GK_)]
            return (pltpu.make_async_copy(ta.at[ia], bufsa[b], sems[b]),
                    pltpu.make_async_copy(tb.at[ib], bufsb[b], sems[2 + b]))

        def w_copy(c, b):
            dst = pl.ds(base + c * GK_, GK_)
            return pltpu.make_async_copy(bufsa[b], gsum.at[dst], sems[4 + b])

        def start(copies):
            for cp in copies:
                cp.start()

        def wait(copies):
            for cp in copies:
                cp.wait()

        start(g_copies(0, 0))

        nlane = D_ // 16

        def chunk_step(c, b):
            nb = 1 - b

            @pl.when(c >= 1)
            def _():
                w_copy(c - 1, nb).wait()

            @pl.when(c + 1 < gnc)
            def _():
                start(g_copies(c + 1, nb))

            wait(g_copies(c, b))

            ba = bufsa[b]
            bb = bufsb[b]

            def add_row(r, carry):
                for l in range(nlane):
                    sl = pl.ds(l * 16, 16)
                    ba[r, sl] = ba[r, sl] + bb[r, sl]
                return carry

            lax.fori_loop(0, GK_, add_row, 0)
            w_copy(c, b).start()

        _ring(gnc, chunk_step)
        w_copy(gnc - 1, (gnc - 1) % 2).wait()

    @functools.partial(
        pl.kernel,
        out_type=jax.ShapeDtypeStruct((NC_, NP_, D_), jnp.float32),
        mesh=mesh,
        scratch_types=[
            pltpu.VMEM((snc, SK_), jnp.int32),
            pltpu.VMEM((SK_, D_), jnp.float32),
            pltpu.VMEM((SK_, D_), jnp.float32),
            pltpu.VMEM((ZR_, D_), jnp.float32),
            pltpu.VMEM_SHARED((NP_, D_), jnp.float32),
            [pltpu.SemaphoreType.DMA] * 4,
        ],
    )
    def _sc_scatter(vals, row2d, out, idx2d, buf0, buf1, zbuf, acc, sems):
        cid = lax.axis_index("c")
        sid = lax.axis_index("s")
        wid = sid * NC_ + cid

        pltpu.sync_copy(row2d.at[wid], idx2d)

        zv = jnp.zeros((16,), jnp.float32)
        nlane = D_ // 16

        def zb(i, carry):
            r = i // nlane
            c = i % nlane
            zbuf[r, pl.ds(c * 16, 16)] = zv
            return carry

        lax.fori_loop(0, ZR_ * nlane, zb, 0)

        def zcopy(j, carry):
            pltpu.sync_copy(zbuf, acc.at[pl.ds(sid * RPT_ + j * ZR_, ZR_)])
            return carry

        lax.fori_loop(0, RPT_ // ZR_, zcopy, 0)
        plsc.subcore_barrier()

        base = wid * epw
        bufs = (buf0, buf1)

        def r_copy(c, b):
            src = vals.at[pl.ds(base + c * SK_, SK_)]
            return pltpu.make_async_copy(src, bufs[b], sems[b])

        def a_copy(c, b):
            return pltpu.make_async_copy(bufs[b], acc.at[idx2d.at[c]],
                                         sems[2 + b])

        r_copy(0, 0).start()

        def chunk_step(c, b):
            nb = 1 - b

            @pl.when(c >= 1)
            def _():
                a_copy(c - 1, nb).wait()

            @pl.when(c + 1 < snc)
            def _():
                r_copy(c + 1, nb).start()

            r_copy(c, b).wait()
            a_copy(c, b).start(add=True)

        _ring(snc, chunk_step)
        a_copy(snc - 1, (snc - 1) % 2).wait()
        plsc.subcore_barrier()

        def drain(j, carry):
            r0 = sid * RPT_ + j * ZR_
            pltpu.sync_copy(acc.at[pl.ds(r0, ZR_)], zbuf)
            pltpu.sync_copy(zbuf, out.at[cid, pl.ds(r0, ZR_)])
            return carry

        lax.fori_loop(0, RPT_ // ZR_, drain, 0)

    return _sc_gather, _sc_scatter


def _edge_body(g_ref, ea_ref, wea_ref, wd_ref, we2_ref, be2_ref,
               wc_ref, bc_ref, out_ref):
    g1 = g_ref[...]
    gm = g1[:, :H_]
    rel = g1[:, H_:H_ + 3]
    d2 = jnp.sum(rel * rel, axis=1, keepdims=True)
    pre = gm + jnp.dot(ea_ref[...], wea_ref[...],
                       preferred_element_type=jnp.float32) + d2 * wd_ref[...]
    m1 = pre * jax.nn.sigmoid(pre)
    m2v = jnp.dot(m1, we2_ref[...], preferred_element_type=jnp.float32) + be2_ref[...]
    m2 = m2v * jax.nn.sigmoid(m2v)
    coef = jnp.tanh(jnp.dot(m2, wc_ref[...],
                            preferred_element_type=jnp.float32) + bc_ref[...])
    z = jnp.zeros((g1.shape[0], PAD_), jnp.float32)
    out_ref[...] = jnp.concatenate([m2, rel * coef, z], axis=1)


def _edges(g, ea, wea, wdr, we2, be2r, wc, bcr):
    e = g.shape[0]
    return pl.pallas_call(
        _edge_body,
        grid=(e // EBLK_,),
        in_specs=[
            pl.BlockSpec((EBLK_, D_), lambda i: (i, 0)),
            pl.BlockSpec((EBLK_, ED_), lambda i: (i, 0)),
            pl.BlockSpec((ED_, H_), lambda i: (0, 0)),
            pl.BlockSpec((1, H_), lambda i: (0, 0)),
            pl.BlockSpec((H_, H_), lambda i: (0, 0)),
            pl.BlockSpec((1, H_), lambda i: (0, 0)),
            pl.BlockSpec((H_, 1), lambda i: (0, 0)),
            pl.BlockSpec((1, 1), lambda i: (0, 0)),
        ],
        out_specs=pl.BlockSpec((EBLK_, D_), lambda i: (i, 0)),
        out_shape=jax.ShapeDtypeStruct((e, D_), jnp.float32),
    )(g, ea, wea, wdr, we2, be2r, wc, bcr)


def _node_body(h_ref, x_ref, p00_ref, p01_ref, p10_ref, p11_ref,
               wn1h_ref, wn1m_ref, bn1_ref, wn2_ref, bn2_ref,
               hn_ref, xn_ref):
    h = h_ref[...]
    p = (p00_ref[...] + p01_ref[...]) + (p10_ref[...] + p11_ref[...])
    magg = p[:, :H_]
    dx = p[:, H_:H_ + 3]
    t = jnp.dot(h, wn1h_ref[...], preferred_element_type=jnp.float32)
    t = t + jnp.dot(magg, wn1m_ref[...], preferred_element_type=jnp.float32)
    t = t + bn1_ref[...]
    t = t * jax.nn.sigmoid(t)
    hn_ref[...] = h + jnp.dot(t, wn2_ref[...],
                              preferred_element_type=jnp.float32) + bn2_ref[...]
    xn_ref[...] = x_ref[...] + dx


def _nodes(h, x, parts, wn1h, wn1m, bn1r, wn2, bn2r):
    g = N_ // NBLK_
    pspec = pl.BlockSpec((NBLK_, D_), lambda i: (i, 0))
    return pl.pallas_call(
        _node_body,
        grid=(g,),
        in_specs=[
            pl.BlockSpec((NBLK_, F_), lambda i: (i, 0)),
            pl.BlockSpec((NBLK_, 3), lambda i: (i, 0)),
            pspec, pspec, pspec, pspec,
            pl.BlockSpec((F_, H_), lambda i: (0, 0)),
            pl.BlockSpec((H_, H_), lambda i: (0, 0)),
            pl.BlockSpec((1, H_), lambda i: (0, 0)),
            pl.BlockSpec((H_, F_), lambda i: (0, 0)),
            pl.BlockSpec((1, F_), lambda i: (0, 0)),
        ],
        out_specs=[
            pl.BlockSpec((NBLK_, F_), lambda i: (i, 0)),
            pl.BlockSpec((NBLK_, 3), lambda i: (i, 0)),
        ],
        out_shape=[
            jax.ShapeDtypeStruct((N_, F_), jnp.float32),
            jax.ShapeDtypeStruct((N_, 3), jnp.float32),
        ],
    )(h, x, *parts, wn1h, wn1m, bn1r, wn2, bn2r)


def kernel(h, x, edge_index, edge_attr, We1, be1, We2, be2, Wc, bc,
           Wn1, bn1, Wn2, bn2):
    row = edge_index[0]
    col = edge_index[1]
    whr = We1[:F_]
    whc = We1[F_:2 * F_]
    wdr = We1[2 * F_:2 * F_ + 1]
    wea = We1[2 * F_ + 1:]
    be2r = be2.reshape(1, H_)
    bcr = bc.reshape(1, 1)

    ta, tb = _tables(h, x, whr, whc, be1.reshape(1, H_))

    parts = []
    off = 0
    for e_chunk in EC_:
        sc_gather, sc_scatter = _sc_kernels(e_chunk)
        rowc = lax.slice(row, (off,), (off + e_chunk,))
        colc = lax.slice(col, (off,), (off + e_chunk,))
        eac = lax.slice(edge_attr, (off, 0), (off + e_chunk, ED_))
        gsum = sc_gather(ta, tb, rowc, colc)
        vals = _edges(gsum, eac, wea, wdr, We2, be2r, Wc, bcr)
        snc = e_chunk // NW_ // SK_
        pc = sc_scatter(vals, rowc.reshape(NW_, snc, SK_))
        parts.extend([pc[0, :N_], pc[1, :N_]])
        off += e_chunk

    return _nodes(h, x, parts, Wn1[:F_], Wn1[F_:],
                  bn1.reshape(1, H_), Wn2, bn2.reshape(1, F_))
